# Initial kernel scaffold; baseline (speedup 1.0000x reference)
import jax
import jax.numpy as jnp
from jax.experimental import pallas as pl


def kernel(x, edge_index, pseudo, W1, R1, b1, W2, R2, b2, W3, R3, b3, Wl1, bl1, Wl2, bl2):
    raise NotImplementedError("write your pallas kernel here")


# speedup vs baseline: 1.0568x; 1.0568x over previous
import functools

import jax
import jax.numpy as jnp
from jax.experimental import pallas as pl
from jax.experimental.pallas import tpu as pltpu

N = 10000
NUM_CLASSES = 6890


def _spline_conv(x, src, dst, pseudo, W, R, b):
    n = x.shape[0]
    ks = 3
    pos = pseudo * (ks - 1.0)
    lo_f = jnp.clip(jnp.floor(pos), 0.0, ks - 2.0)
    frac = pos - lo_f
    lo = lo_f.astype(jnp.int32)
    xW = jnp.einsum('ni,kio->nko', x, W)
    out_e = jnp.zeros((src.shape[0], W.shape[2]), dtype=x.dtype)
    for b0 in (0, 1):
        for b1 in (0, 1):
            for b2 in (0, 1):
                idx = (lo[:, 0] + b0) * 9 + (lo[:, 1] + b1) * 3 + (lo[:, 2] + b2)
                w = ((frac[:, 0] if b0 else 1.0 - frac[:, 0])
                     * (frac[:, 1] if b1 else 1.0 - frac[:, 1])
                     * (frac[:, 2] if b2 else 1.0 - frac[:, 2]))
                out_e = out_e + w[:, None] * xW[src, idx]
    agg = jax.ops.segment_sum(out_e, dst, num_segments=n)
    deg = jax.ops.segment_sum(jnp.ones((src.shape[0],), dtype=x.dtype), dst, num_segments=n)
    agg = agg / jnp.clip(deg, 1.0)[:, None]
    return agg + x @ R + b


def _head_body(h_ref, wl1_ref, bl1_ref, wl2_ref, bl2_ref, out_ref):
    h = h_ref[...]
    t = jnp.dot(h, wl1_ref[...], preferred_element_type=jnp.float32)
    t = jax.nn.relu(t + bl1_ref[...])
    logits = jnp.dot(t.astype(jnp.bfloat16), wl2_ref[...],
                     preferred_element_type=jnp.float32)
    logits = logits + bl2_ref[...]
    m = jnp.max(logits, axis=-1, keepdims=True)
    lse = jnp.log(jnp.sum(jnp.exp(logits - m), axis=-1, keepdims=True))
    out_ref[...] = logits - m - lse


@jax.jit
def _head(h3, Wl1, bl1, Wl2, bl2):
    BM = 400
    grid = (N // BM,)
    return pl.pallas_call(
        _head_body,
        grid=grid,
        in_specs=[
            pl.BlockSpec((BM, 128), lambda i: (i, 0)),
            pl.BlockSpec((128, 256), lambda i: (0, 0)),
            pl.BlockSpec((1, 256), lambda i: (0, 0)),
            pl.BlockSpec((256, NUM_CLASSES), lambda i: (0, 0)),
            pl.BlockSpec((1, NUM_CLASSES), lambda i: (0, 0)),
        ],
        out_specs=pl.BlockSpec((BM, NUM_CLASSES), lambda i: (i, 0)),
        out_shape=jax.ShapeDtypeStruct((N, NUM_CLASSES), jnp.float32),
    )(h3, Wl1, bl1.reshape(1, -1), Wl2.astype(jnp.bfloat16), bl2.reshape(1, -1))


def kernel(x, edge_index, pseudo, W1, R1, b1, W2, R2, b2, W3, R3, b3, Wl1, bl1, Wl2, bl2):
    src = edge_index[0]
    dst = edge_index[1]
    h = jax.nn.relu(_spline_conv(x, src, dst, pseudo, W1, R1, b1))
    h = jax.nn.relu(_spline_conv(h, src, dst, pseudo, W2, R2, b2))
    h = jax.nn.relu(_spline_conv(h, src, dst, pseudo, W3, R3, b3))
    return _head(h, Wl1, bl1, Wl2, bl2)
